# baseline (speedup 1.0000x reference)
import functools

import jax
import jax.numpy as jnp
from jax import lax
from jax.experimental import pallas as pl
from jax.experimental.pallas import tpu as pltpu
from jax.experimental.pallas import tpu_sc as plsc

N = 10000
E = 320000
F_IN = 128
F_HID = 256
F_HALF = 128
N_GR = 128
N_CL = 10

NB = 25
BN_ = N // NB

N_TILES = 16
CH = 64
NCH = 320
E_TILE = NCH * CH
EP = N_TILES * E_TILE
NPAD = 10240


@functools.cache
def _build_agg_sc():
  mesh = plsc.VectorSubcoreMesh(core_axis_name="c", subcore_axis_name="s")

  @functools.partial(
    pl.kernel, mesh=mesh,
    out_type=jax.ShapeDtypeStruct((2, NPAD, F_HALF), jnp.float32),
    scratch_types=[
        pltpu.VMEM((16 * CH,), jnp.int32),
        pltpu.VMEM((16, CH), jnp.int32),
        pltpu.VMEM((CH, F_HALF), jnp.float32),
        pltpu.VMEM((CH, F_HALF), jnp.float32),
        pltpu.VMEM((CH, F_HALF), jnp.float32),
        pltpu.VMEM((CH, F_HALF), jnp.float32),
        pltpu.VMEM((32, F_HALF), jnp.float32),
        pltpu.VMEM_SHARED((NPAD, F_HALF), jnp.float32),
        pltpu.SemaphoreType.DMA,
        pltpu.SemaphoreType.DMA,
        pltpu.SemaphoreType.DMA,
        pltpu.SemaphoreType.DMA,
    ],
  )
  def _agg_sc(hp_hbm, row_hbm, col_hbm, out_hbm, rowb, colb, gbuf0, gbuf1,
              gbuf2, gbuf3, zbuf, acc, sem0, sem1, sem2, sem3):
    c = lax.axis_index("c")
    s = lax.axis_index("s")
    zero16 = jnp.zeros((16,), jnp.float32)

    def _z(i, _):
        zbuf[i // 8, pl.ds((i % 8) * 16, 16)] = zero16
        return 0
    lax.fori_loop(0, 32 * (F_HALF // 16), _z, 0)

    def _zs(t, _):
        pltpu.sync_copy(zbuf, acc.at[pl.ds(s * 640 + t * 32, 32)])
        return 0
    lax.fori_loop(0, 20, _zs, 0)

    plsc.subcore_barrier()

    def _blk(b, _):
        base = b * (16 * CH)
        pltpu.sync_copy(row_hbm.at[c, s, 0, pl.ds(base, 16 * CH)], rowb)
        pltpu.sync_copy(col_hbm.at[s, pl.ds(b * 16, 16), :], colb)

        bufs = (gbuf0, gbuf1, gbuf2, gbuf3)
        sems = (sem0, sem1, sem2, sem3)

        def _fire(k, gbuf, sem):
            return pltpu.async_copy(
                hp_hbm.at[rowb.at[pl.ds(k * CH, CH)]], gbuf, sem)

        gh = [_fire(k, bufs[k], sems[k]) for k in range(4)]
        for k in range(16):
            gh[k % 4].wait()
            pltpu.sync_copy(bufs[k % 4], acc.at[colb.at[k]], add=True)
            if k + 4 < 16:
                gh[(k + 4) % 4] = _fire(k + 4, bufs[k % 4], sems[k % 4])
        return 0
    lax.fori_loop(0, NCH // 16, _blk, 0)

    plsc.subcore_barrier()

    def _wb(t, _):
        r0 = s * 640 + t * 32
        pltpu.sync_copy(acc.at[pl.ds(r0, 32)], zbuf)
        pltpu.sync_copy(zbuf, out_hbm.at[c, pl.ds(r0, 32), :])
        return 0
    lax.fori_loop(0, 20, _wb, 0)

  return _agg_sc


def _prep_body(x_ref, w_ref, degp_ref, h_ref, hp_ref, dinv_ref):
    deg_col = degp_ref[0, :, 0:1] + 1.0
    dinv = lax.rsqrt(deg_col)
    dinv_ref[...] = jnp.broadcast_to(dinv, (BN_, F_HALF))
    h = jnp.dot(x_ref[...], w_ref[...], preferred_element_type=jnp.float32)
    h_ref[...] = h
    hp = h * dinv
    hp_ref[0] = hp[:, :F_HALF]
    hp_ref[1] = hp[:, F_HALF:]


def _prep_tc(x, w1, degp):
    return pl.pallas_call(
        _prep_body,
        grid=(NB,),
        in_specs=[
            pl.BlockSpec((BN_, F_IN), lambda i: (i, 0)),
            pl.BlockSpec((F_IN, F_HID), lambda i: (0, 0)),
            pl.BlockSpec((1, BN_, F_HALF), lambda i: (0, i, 0)),
        ],
        out_specs=[
            pl.BlockSpec((BN_, F_HID), lambda i: (i, 0)),
            pl.BlockSpec((2, BN_, F_HALF), lambda i: (0, i, 0)),
            pl.BlockSpec((BN_, F_HALF), lambda i: (i, 0)),
        ],
        out_shape=[
            jax.ShapeDtypeStruct((N, F_HID), jnp.float32),
            jax.ShapeDtypeStruct((2, N, F_HALF), jnp.float32),
            jax.ShapeDtypeStruct((N, F_HALF), jnp.float32),
        ],
    )(x, w1, degp)


def _post_body(agg_ref, h_ref, dinv_ref, b_ref, z_ref, ssum_ref, ssq_ref):
    d = dinv_ref[:, 0:1]
    a = jnp.concatenate([agg_ref[0], agg_ref[1]], axis=-1)
    z = d * a + (d * d) * h_ref[...] + b_ref[...]
    z_ref[...] = z

    @pl.when(pl.program_id(0) == 0)
    def _init():
        ssum_ref[...] = jnp.zeros_like(ssum_ref)
        ssq_ref[...] = jnp.zeros_like(ssq_ref)

    ssum_ref[...] += jnp.sum(z, axis=0, keepdims=True)
    ssq_ref[...] += jnp.sum(z * z, axis=0, keepdims=True)


def _post_tc(agg, h, dinvb, br):
    return pl.pallas_call(
        _post_body,
        grid=(NB,),
        in_specs=[
            pl.BlockSpec((2, BN_, F_HALF), lambda i: (0, i, 0)),
            pl.BlockSpec((BN_, F_HID), lambda i: (i, 0)),
            pl.BlockSpec((BN_, F_HALF), lambda i: (i, 0)),
            pl.BlockSpec((1, F_HID), lambda i: (0, 0)),
        ],
        out_specs=[
            pl.BlockSpec((BN_, F_HID), lambda i: (i, 0)),
            pl.BlockSpec((1, F_HID), lambda i: (0, 0)),
            pl.BlockSpec((1, F_HID), lambda i: (0, 0)),
        ],
        out_shape=[
            jax.ShapeDtypeStruct((N, F_HID), jnp.float32),
            jax.ShapeDtypeStruct((1, F_HID), jnp.float32),
            jax.ShapeDtypeStruct((1, F_HID), jnp.float32),
        ],
    )(agg, h, dinvb, br)


def _bn_relu(z_ref, ssum_ref, ssq_ref, g_ref, be_ref):
    mean = ssum_ref[...] * (1.0 / N)
    var = ssq_ref[...] * (1.0 / N) - mean * mean
    xn = (z_ref[...] - mean) * lax.rsqrt(var + 1e-5) * g_ref[...] + be_ref[...]
    return jnp.maximum(xn, 0.0)


def _bnmat_body(z_ref, ssum_ref, ssq_ref, g_ref, be_ref, dinv_ref, w_ref,
                hpre_ref, hp_ref):
    hrelu = _bn_relu(z_ref, ssum_ref, ssq_ref, g_ref, be_ref)
    hpre = jnp.dot(hrelu, w_ref[...], preferred_element_type=jnp.float32)
    hpre_ref[...] = hpre
    hp = hpre * dinv_ref[:, 0:1]
    hp_ref[0] = hp[:, :F_HALF]
    hp_ref[1] = hp[:, F_HALF:]


def _bnmat_tc(z, ssum, ssq, gr, ber, dinvb, w):
    return pl.pallas_call(
        _bnmat_body,
        grid=(NB,),
        in_specs=[
            pl.BlockSpec((BN_, F_HID), lambda i: (i, 0)),
            pl.BlockSpec((1, F_HID), lambda i: (0, 0)),
            pl.BlockSpec((1, F_HID), lambda i: (0, 0)),
            pl.BlockSpec((1, F_HID), lambda i: (0, 0)),
            pl.BlockSpec((1, F_HID), lambda i: (0, 0)),
            pl.BlockSpec((BN_, F_HALF), lambda i: (i, 0)),
            pl.BlockSpec((F_HID, F_HID), lambda i: (0, 0)),
        ],
        out_specs=[
            pl.BlockSpec((BN_, F_HID), lambda i: (i, 0)),
            pl.BlockSpec((2, BN_, F_HALF), lambda i: (0, i, 0)),
        ],
        out_shape=[
            jax.ShapeDtypeStruct((N, F_HID), jnp.float32),
            jax.ShapeDtypeStruct((2, N, F_HALF), jnp.float32),
        ],
    )(z, ssum, ssq, gr, ber, dinvb, w)


def _pool_body(z_ref, ssum_ref, ssq_ref, g_ref, be_ref, batch_ref, wc_ref,
               bc_ref, logits_ref, acc_ref):
    hrelu = _bn_relu(z_ref, ssum_ref, ssq_ref, g_ref, be_ref)
    bb = batch_ref[0, 0, :]
    gi = lax.broadcasted_iota(jnp.int32, (N_GR, 1), 0)
    oh = (gi == bb[None, :]).astype(jnp.float32)
    part = jnp.dot(oh, hrelu, preferred_element_type=jnp.float32)

    @pl.when(pl.program_id(0) == 0)
    def _init():
        acc_ref[...] = jnp.zeros_like(acc_ref)

    acc_ref[...] += part

    @pl.when(pl.program_id(0) == NB - 1)
    def _fin():
        logits_ref[...] = jnp.dot(
            acc_ref[...], wc_ref[...],
            preferred_element_type=jnp.float32) + bc_ref[...]


def _pool_tc(z, ssum, ssq, gr, ber, batch3, wc, bcr):
    return pl.pallas_call(
        _pool_body,
        grid=(NB,),
        in_specs=[
            pl.BlockSpec((BN_, F_HID), lambda i: (i, 0)),
            pl.BlockSpec((1, F_HID), lambda i: (0, 0)),
            pl.BlockSpec((1, F_HID), lambda i: (0, 0)),
            pl.BlockSpec((1, F_HID), lambda i: (0, 0)),
            pl.BlockSpec((1, F_HID), lambda i: (0, 0)),
            pl.BlockSpec((1, 1, BN_), lambda i: (i, 0, 0)),
            pl.BlockSpec((F_HID, N_CL), lambda i: (0, 0)),
            pl.BlockSpec((1, N_CL), lambda i: (0, 0)),
        ],
        out_specs=pl.BlockSpec((N_GR, N_CL), lambda i: (0, 0)),
        out_shape=jax.ShapeDtypeStruct((N_GR, N_CL), jnp.float32),
        scratch_shapes=[pltpu.VMEM((N_GR, F_HID), jnp.float32)],
    )(z, ssum, ssq, gr, ber, batch3, wc, bcr)


def kernel(x, edge_index, batch, W1, b1, g1, be1, W2, b2, g2, be2,
           W3, b3, g3, be3, Wc, bc):
    row = edge_index[0]
    col = edge_index[1]
    rowp = jnp.concatenate([row, jnp.zeros((EP - E,), jnp.int32)])
    colp = jnp.concatenate([col, jnp.full((EP - E,), NPAD - 1, jnp.int32)])
    row_agg = jnp.stack([rowp, rowp + N]).reshape(2, N_TILES, 1, E_TILE)
    col_agg = colp.reshape(N_TILES, NCH, CH)
    batch3 = batch.reshape(NB, 1, BN_)
    b1r, g1r, be1r = b1.reshape(1, -1), g1.reshape(1, -1), be1.reshape(1, -1)
    b2r, g2r, be2r = b2.reshape(1, -1), g2.reshape(1, -1), be2.reshape(1, -1)
    b3r, g3r, be3r = b3.reshape(1, -1), g3.reshape(1, -1), be3.reshape(1, -1)
    bcr = bc.reshape(1, -1)

    ones_tab = jnp.ones((2 * N, F_HALF), jnp.float32)
    agg_sc = _build_agg_sc()
    degp = agg_sc(ones_tab, row_agg, col_agg)
    h1, hp1, dinvb = _prep_tc(x, W1, degp)
    agg1 = agg_sc(hp1.reshape(2 * N, F_HALF), row_agg, col_agg)
    z1, s1, q1 = _post_tc(agg1, h1, dinvb, b1r)
    h2, hp2 = _bnmat_tc(z1, s1, q1, g1r, be1r, dinvb, W2)
    agg2 = agg_sc(hp2.reshape(2 * N, F_HALF), row_agg, col_agg)
    z2, s2, q2 = _post_tc(agg2, h2, dinvb, b2r)
    h3, hp3 = _bnmat_tc(z2, s2, q2, g2r, be2r, dinvb, W3)
    agg3 = agg_sc(hp3.reshape(2 * N, F_HALF), row_agg, col_agg)
    z3, s3, q3 = _post_tc(agg3, h3, dinvb, b3r)
    return _pool_tc(z3, s3, q3, g3r, be3r, batch3, Wc, bcr)


# speedup vs baseline: 1.3550x; 1.3550x over previous
import functools

import jax
import jax.numpy as jnp
from jax import lax
from jax.experimental import pallas as pl
from jax.experimental.pallas import tpu as pltpu
from jax.experimental.pallas import tpu_sc as plsc

N = 10000
E = 320000
F_IN = 128
F_HID = 256
F_HALF = 128
N_GR = 128
N_CL = 10

NB = 25
BN_ = N // NB

N_TILES = 16
CH = 64
NCH = 320
E_TILE = NCH * CH
EP = N_TILES * E_TILE
NPAD = 10240


@functools.cache
def _build_agg_sc():
  mesh = plsc.VectorSubcoreMesh(core_axis_name="c", subcore_axis_name="s")

  @functools.partial(
    pl.kernel, mesh=mesh,
    out_type=jax.ShapeDtypeStruct((2, NPAD, F_HALF), jnp.float32),
    scratch_types=[
        pltpu.VMEM((16 * CH,), jnp.int32),
        pltpu.VMEM((16, CH), jnp.int32),
        pltpu.VMEM((CH, F_HALF), jnp.float32),
        pltpu.VMEM((CH, F_HALF), jnp.float32),
        pltpu.VMEM((CH, F_HALF), jnp.float32),
        pltpu.VMEM((CH, F_HALF), jnp.float32),
        pltpu.VMEM((16, F_HALF), jnp.float32),
        pltpu.VMEM_SHARED((NPAD, F_HALF), jnp.float32),
        pltpu.SemaphoreType.DMA,
        pltpu.SemaphoreType.DMA,
        pltpu.SemaphoreType.DMA,
        pltpu.SemaphoreType.DMA,
    ],
  )
  def _agg_sc(hp_hbm, row_hbm, col_hbm, out_hbm, rowb, colb, gbuf0, gbuf1,
              gbuf2, gbuf3, zbuf, acc, sem0, sem1, sem2, sem3):
    c = lax.axis_index("c")
    s = lax.axis_index("s")
    zero16 = jnp.zeros((16,), jnp.float32)

    def _z(i, _):
        zbuf[i // 8, pl.ds((i % 8) * 16, 16)] = zero16
        return 0
    lax.fori_loop(0, 16 * (F_HALF // 16), _z, 0)

    def _zs(t, _):
        pltpu.sync_copy(zbuf, acc.at[pl.ds(s * 640 + t * 16, 16)])
        return 0
    lax.fori_loop(0, 40, _zs, 0)

    plsc.subcore_barrier()

    def _blk(b, _):
        base = b * (16 * CH)
        pltpu.sync_copy(row_hbm.at[c, s, 0, pl.ds(base, 16 * CH)], rowb)
        pltpu.sync_copy(col_hbm.at[s, pl.ds(b * 16, 16), :], colb)

        bufs = (gbuf0, gbuf1, gbuf2, gbuf3)
        sems = (sem0, sem1, sem2, sem3)

        def _fire(k, gbuf, sem):
            return pltpu.async_copy(
                hp_hbm.at[rowb.at[pl.ds(k * CH, CH)]], gbuf, sem)

        gh = [_fire(k, bufs[k], sems[k]) for k in range(4)]
        for k in range(16):
            gh[k % 4].wait()
            pltpu.sync_copy(bufs[k % 4], acc.at[colb.at[k]], add=True)
            if k + 4 < 16:
                gh[(k + 4) % 4] = _fire(k + 4, bufs[k % 4], sems[k % 4])
        return 0
    lax.fori_loop(0, NCH // 16, _blk, 0)

    plsc.subcore_barrier()

    def _wb(t, _):
        r0 = s * 640 + t * 16
        pltpu.sync_copy(acc.at[pl.ds(r0, 16)], zbuf)
        pltpu.sync_copy(zbuf, out_hbm.at[c, pl.ds(r0, 16), :])
        return 0
    lax.fori_loop(0, 40, _wb, 0)

  return _agg_sc


@functools.cache
def _build_deg_sc():
  mesh = plsc.VectorSubcoreMesh(core_axis_name="c", subcore_axis_name="s")

  @functools.partial(
    pl.kernel, mesh=mesh,
    out_type=jax.ShapeDtypeStruct((NPAD, 16), jnp.float32),
    scratch_types=[
        pltpu.VMEM((16, CH), jnp.int32),
        pltpu.VMEM((CH, 16), jnp.float32),
        pltpu.VMEM((64, 16), jnp.float32),
        pltpu.VMEM_SHARED((NPAD, 16), jnp.float32),
    ],
  )
  def _deg_sc(col_hbm, out_hbm, colb, onesb, zbuf, acc):
    c = lax.axis_index("c")
    s = lax.axis_index("s")
    zero16 = jnp.zeros((16,), jnp.float32)
    ones16 = jnp.ones((16,), jnp.float32)

    def _z(i, _):
        zbuf[i, pl.ds(0, 16)] = zero16
        return 0
    lax.fori_loop(0, 64, _z, 0)

    def _o(i, _):
        onesb[i, pl.ds(0, 16)] = ones16
        return 0
    lax.fori_loop(0, CH, _o, 0)

    def _zs(t, _):
        pltpu.sync_copy(zbuf, acc.at[pl.ds(s * 640 + t * 64, 64)])
        return 0
    lax.fori_loop(0, 10, _zs, 0)

    plsc.subcore_barrier()

    def _blk(b, _):
        pltpu.sync_copy(col_hbm.at[s, pl.ds(b * 16, 16), :], colb)
        for k in range(16):
            pltpu.sync_copy(onesb, acc.at[colb.at[k]], add=True)
        return 0
    lax.fori_loop(0, NCH // 16, _blk, 0)

    plsc.subcore_barrier()

    @pl.when(c == 0)
    def _wb():
        def _w(t, _):
            r0 = s * 640 + t * 64
            pltpu.sync_copy(acc.at[pl.ds(r0, 64)], zbuf)
            pltpu.sync_copy(zbuf, out_hbm.at[pl.ds(r0, 64), :])
            return 0
        lax.fori_loop(0, 10, _w, 0)

  return _deg_sc


def _prep_body(x_ref, w_ref, degp_ref, h_ref, hp_ref, dinv_ref):
    deg_col = degp_ref[:, 0:1] + 1.0
    dinv = lax.rsqrt(deg_col)
    dinv_ref[...] = jnp.broadcast_to(dinv, (BN_, F_HALF))
    h = jnp.dot(x_ref[...], w_ref[...], preferred_element_type=jnp.float32)
    h_ref[...] = h
    hp = h * dinv
    hp_ref[0] = hp[:, :F_HALF]
    hp_ref[1] = hp[:, F_HALF:]


def _prep_tc(x, w1, degp):
    return pl.pallas_call(
        _prep_body,
        grid=(NB,),
        in_specs=[
            pl.BlockSpec((BN_, F_IN), lambda i: (i, 0)),
            pl.BlockSpec((F_IN, F_HID), lambda i: (0, 0)),
            pl.BlockSpec((BN_, 16), lambda i: (i, 0)),
        ],
        out_specs=[
            pl.BlockSpec((BN_, F_HID), lambda i: (i, 0)),
            pl.BlockSpec((2, BN_, F_HALF), lambda i: (0, i, 0)),
            pl.BlockSpec((BN_, F_HALF), lambda i: (i, 0)),
        ],
        out_shape=[
            jax.ShapeDtypeStruct((N, F_HID), jnp.float32),
            jax.ShapeDtypeStruct((2, N, F_HALF), jnp.float32),
            jax.ShapeDtypeStruct((N, F_HALF), jnp.float32),
        ],
    )(x, w1, degp)


def _post_body(agg_ref, h_ref, dinv_ref, b_ref, z_ref, ssum_ref, ssq_ref):
    d = dinv_ref[:, 0:1]
    a = jnp.concatenate([agg_ref[0], agg_ref[1]], axis=-1)
    z = d * a + (d * d) * h_ref[...] + b_ref[...]
    z_ref[...] = z

    @pl.when(pl.program_id(0) == 0)
    def _init():
        ssum_ref[...] = jnp.zeros_like(ssum_ref)
        ssq_ref[...] = jnp.zeros_like(ssq_ref)

    ssum_ref[...] += jnp.sum(z, axis=0, keepdims=True)
    ssq_ref[...] += jnp.sum(z * z, axis=0, keepdims=True)


def _post_tc(agg, h, dinvb, br):
    return pl.pallas_call(
        _post_body,
        grid=(NB,),
        in_specs=[
            pl.BlockSpec((2, BN_, F_HALF), lambda i: (0, i, 0)),
            pl.BlockSpec((BN_, F_HID), lambda i: (i, 0)),
            pl.BlockSpec((BN_, F_HALF), lambda i: (i, 0)),
            pl.BlockSpec((1, F_HID), lambda i: (0, 0)),
        ],
        out_specs=[
            pl.BlockSpec((BN_, F_HID), lambda i: (i, 0)),
            pl.BlockSpec((1, F_HID), lambda i: (0, 0)),
            pl.BlockSpec((1, F_HID), lambda i: (0, 0)),
        ],
        out_shape=[
            jax.ShapeDtypeStruct((N, F_HID), jnp.float32),
            jax.ShapeDtypeStruct((1, F_HID), jnp.float32),
            jax.ShapeDtypeStruct((1, F_HID), jnp.float32),
        ],
    )(agg, h, dinvb, br)


def _bn_relu(z_ref, ssum_ref, ssq_ref, g_ref, be_ref):
    mean = ssum_ref[...] * (1.0 / N)
    var = ssq_ref[...] * (1.0 / N) - mean * mean
    xn = (z_ref[...] - mean) * lax.rsqrt(var + 1e-5) * g_ref[...] + be_ref[...]
    return jnp.maximum(xn, 0.0)


def _bnmat_body(z_ref, ssum_ref, ssq_ref, g_ref, be_ref, dinv_ref, w_ref,
                hpre_ref, hp_ref):
    hrelu = _bn_relu(z_ref, ssum_ref, ssq_ref, g_ref, be_ref)
    hpre = jnp.dot(hrelu, w_ref[...], preferred_element_type=jnp.float32)
    hpre_ref[...] = hpre
    hp = hpre * dinv_ref[:, 0:1]
    hp_ref[0] = hp[:, :F_HALF]
    hp_ref[1] = hp[:, F_HALF:]


def _bnmat_tc(z, ssum, ssq, gr, ber, dinvb, w):
    return pl.pallas_call(
        _bnmat_body,
        grid=(NB,),
        in_specs=[
            pl.BlockSpec((BN_, F_HID), lambda i: (i, 0)),
            pl.BlockSpec((1, F_HID), lambda i: (0, 0)),
            pl.BlockSpec((1, F_HID), lambda i: (0, 0)),
            pl.BlockSpec((1, F_HID), lambda i: (0, 0)),
            pl.BlockSpec((1, F_HID), lambda i: (0, 0)),
            pl.BlockSpec((BN_, F_HALF), lambda i: (i, 0)),
            pl.BlockSpec((F_HID, F_HID), lambda i: (0, 0)),
        ],
        out_specs=[
            pl.BlockSpec((BN_, F_HID), lambda i: (i, 0)),
            pl.BlockSpec((2, BN_, F_HALF), lambda i: (0, i, 0)),
        ],
        out_shape=[
            jax.ShapeDtypeStruct((N, F_HID), jnp.float32),
            jax.ShapeDtypeStruct((2, N, F_HALF), jnp.float32),
        ],
    )(z, ssum, ssq, gr, ber, dinvb, w)


def _pool_body(z_ref, ssum_ref, ssq_ref, g_ref, be_ref, batch_ref, wc_ref,
               bc_ref, logits_ref, acc_ref):
    hrelu = _bn_relu(z_ref, ssum_ref, ssq_ref, g_ref, be_ref)
    bb = batch_ref[0, 0, :]
    gi = lax.broadcasted_iota(jnp.int32, (N_GR, 1), 0)
    oh = (gi == bb[None, :]).astype(jnp.float32)
    part = jnp.dot(oh, hrelu, preferred_element_type=jnp.float32)

    @pl.when(pl.program_id(0) == 0)
    def _init():
        acc_ref[...] = jnp.zeros_like(acc_ref)

    acc_ref[...] += part

    @pl.when(pl.program_id(0) == NB - 1)
    def _fin():
        logits_ref[...] = jnp.dot(
            acc_ref[...], wc_ref[...],
            preferred_element_type=jnp.float32) + bc_ref[...]


def _pool_tc(z, ssum, ssq, gr, ber, batch3, wc, bcr):
    return pl.pallas_call(
        _pool_body,
        grid=(NB,),
        in_specs=[
            pl.BlockSpec((BN_, F_HID), lambda i: (i, 0)),
            pl.BlockSpec((1, F_HID), lambda i: (0, 0)),
            pl.BlockSpec((1, F_HID), lambda i: (0, 0)),
            pl.BlockSpec((1, F_HID), lambda i: (0, 0)),
            pl.BlockSpec((1, F_HID), lambda i: (0, 0)),
            pl.BlockSpec((1, 1, BN_), lambda i: (i, 0, 0)),
            pl.BlockSpec((F_HID, N_CL), lambda i: (0, 0)),
            pl.BlockSpec((1, N_CL), lambda i: (0, 0)),
        ],
        out_specs=pl.BlockSpec((N_GR, N_CL), lambda i: (0, 0)),
        out_shape=jax.ShapeDtypeStruct((N_GR, N_CL), jnp.float32),
        scratch_shapes=[pltpu.VMEM((N_GR, F_HID), jnp.float32)],
    )(z, ssum, ssq, gr, ber, batch3, wc, bcr)


def kernel(x, edge_index, batch, W1, b1, g1, be1, W2, b2, g2, be2,
           W3, b3, g3, be3, Wc, bc):
    row = edge_index[0]
    col = edge_index[1]
    rowp = jnp.concatenate([row, jnp.zeros((EP - E,), jnp.int32)])
    colp = jnp.concatenate([col, jnp.full((EP - E,), NPAD - 1, jnp.int32)])
    row_agg = jnp.stack([rowp, rowp + N]).reshape(2, N_TILES, 1, E_TILE)
    col_agg = colp.reshape(N_TILES, NCH, CH)
    batch3 = batch.reshape(NB, 1, BN_)
    b1r, g1r, be1r = b1.reshape(1, -1), g1.reshape(1, -1), be1.reshape(1, -1)
    b2r, g2r, be2r = b2.reshape(1, -1), g2.reshape(1, -1), be2.reshape(1, -1)
    b3r, g3r, be3r = b3.reshape(1, -1), g3.reshape(1, -1), be3.reshape(1, -1)
    bcr = bc.reshape(1, -1)

    agg_sc = _build_agg_sc()
    degp = _build_deg_sc()(col_agg)
    h1, hp1, dinvb = _prep_tc(x, W1, degp)
    agg1 = agg_sc(hp1.reshape(2 * N, F_HALF), row_agg, col_agg)
    z1, s1, q1 = _post_tc(agg1, h1, dinvb, b1r)
    h2, hp2 = _bnmat_tc(z1, s1, q1, g1r, be1r, dinvb, W2)
    agg2 = agg_sc(hp2.reshape(2 * N, F_HALF), row_agg, col_agg)
    z2, s2, q2 = _post_tc(agg2, h2, dinvb, b2r)
    h3, hp3 = _bnmat_tc(z2, s2, q2, g2r, be2r, dinvb, W3)
    agg3 = agg_sc(hp3.reshape(2 * N, F_HALF), row_agg, col_agg)
    z3, s3, q3 = _post_tc(agg3, h3, dinvb, b3r)
    return _pool_tc(z3, s3, q3, g3r, be3r, batch3, Wc, bcr)
